# baseline (speedup 1.0000x reference)
import functools

import jax
import jax.numpy as jnp
from jax import lax
from jax.experimental import pallas as pl
from jax.experimental.pallas import tpu as pltpu
from jax.experimental.pallas import tpu_sc as plsc

_LANES = 16


def _gather_body(n_j, row_len, x_hbm, i1_hbm, i2_hbm, out_hbm,
                 i1_v, i2_v, idx_v, rows_v, sem):
    cid = lax.axis_index("c")
    sid = lax.axis_index("s")

    @pl.when(jnp.logical_and(cid == 0, sid == 0))
    def _():
        pltpu.sync_copy(i1_hbm, i1_v)
        pltpu.sync_copy(i2_hbm, i2_v)
        lane = lax.iota(jnp.int32, _LANES)
        j = lax.rem(lane, jnp.int32(n_j))
        r1 = i1_v[...]
        dnums = lax.GatherDimensionNumbers(
            offset_dims=(), collapsed_slice_dims=(0,), start_index_map=(0,))
        r2 = lax.gather(i2_v[...], j[:, None], dnums, slice_sizes=(1,),
                        mode=lax.GatherScatterMode.PROMISE_IN_BOUNDS)
        idx_v[...] = r1 * jnp.int32(row_len) + r2
        pltpu.async_copy(x_hbm.at[idx_v], rows_v, sem).wait()
        pltpu.sync_copy(rows_v, out_hbm)


def kernel(x, index1, index2):
    h, w, d = x.shape
    n_i, n_j = index1.shape
    n = n_i * n_j
    x2d = x.reshape(h * w, d)
    i1 = jnp.pad(index1.reshape(-1).astype(jnp.int32), (0, _LANES - n))
    i2 = jnp.pad(index2.astype(jnp.int32), (0, _LANES - index2.shape[0]))

    mesh = plsc.VectorSubcoreMesh(core_axis_name="c", subcore_axis_name="s")
    run = pl.kernel(
        functools.partial(_gather_body, n_j, w),
        out_type=jax.ShapeDtypeStruct((_LANES, d), jnp.float32),
        mesh=mesh,
        compiler_params=pltpu.CompilerParams(use_tc_tiling_on_sc=False),
        scratch_types=[
            pltpu.VMEM((_LANES,), jnp.int32),
            pltpu.VMEM((_LANES,), jnp.int32),
            pltpu.VMEM((_LANES,), jnp.int32),
            pltpu.VMEM((_LANES, d), jnp.float32),
            pltpu.SemaphoreType.DMA,
        ],
    )
    out16 = run(x2d, i1, i2)
    return out16[:n].reshape(n_i, n_j, d)


# speedup vs baseline: 1.4649x; 1.4649x over previous
import functools

import jax
import jax.numpy as jnp
from jax import lax
from jax.experimental import pallas as pl
from jax.experimental.pallas import tpu as pltpu
from jax.experimental.pallas import tpu_sc as plsc

_LANES = 16


def _gather_body(n, n_j, x_hbm, i1_hbm, i2_hbm, out_hbm,
                 i1_v, i2_v, rows_v, sem):
    cid = lax.axis_index("c")
    sid = lax.axis_index("s")

    @pl.when(jnp.logical_and(cid == 0, sid == 0))
    def _():
        pltpu.sync_copy(i1_hbm, i1_v)
        pltpu.sync_copy(i2_hbm, i2_v)
        vh = i1_v[...]
        vw = i2_v[...]
        descs = []
        for k in range(n):
            hk = vh[k]
            wk = vw[k % n_j]
            descs.append(pltpu.async_copy(x_hbm.at[hk, wk], rows_v.at[k], sem))
        for d in descs:
            d.wait()
        pltpu.sync_copy(rows_v, out_hbm)


def kernel(x, index1, index2):
    h, w, d = x.shape
    n_i, n_j = index1.shape
    n = n_i * n_j
    i1 = jnp.pad(index1.reshape(-1).astype(jnp.int32), (0, _LANES - n))
    i2 = jnp.pad(index2.astype(jnp.int32), (0, _LANES - index2.shape[0]))

    mesh = plsc.VectorSubcoreMesh(core_axis_name="c", subcore_axis_name="s")
    run = pl.kernel(
        functools.partial(_gather_body, n, n_j),
        out_type=jax.ShapeDtypeStruct((_LANES, d), jnp.float32),
        mesh=mesh,
        scratch_types=[
            pltpu.VMEM((_LANES,), jnp.int32),
            pltpu.VMEM((_LANES,), jnp.int32),
            pltpu.VMEM((_LANES, d), jnp.float32),
            pltpu.SemaphoreType.DMA,
        ],
    )
    out16 = run(x, i1, i2)
    return out16[:n].reshape(n_i, n_j, d)


# speedup vs baseline: 122.4233x; 83.5727x over previous
import functools

import jax
import jax.numpy as jnp
from jax import lax
from jax.experimental import pallas as pl
from jax.experimental.pallas import tpu as pltpu
from jax.experimental.pallas import tpu_sc as plsc

_LANES = 16
_BLK = 128

_DNUMS = lax.GatherDimensionNumbers(
    offset_dims=(), collapsed_slice_dims=(0,), start_index_map=(0,))


def _lane_splat(vec, idx):
    return lax.gather(vec, idx[:, None], _DNUMS, slice_sizes=(1,),
                      mode=lax.GatherScatterMode.PROMISE_IN_BOUNDS)


def _gather_body(n, n_j, d, x_hbm, i1_hbm, i2_hbm, out_hbm,
                 i1_v, i2_v, blk_v, col_v):
    cid = lax.axis_index("c")
    sid = lax.axis_index("s")
    lane = lax.iota(jnp.int32, _LANES)

    for k in range(n):

        @pl.when(jnp.logical_and(cid == 0, sid == k))
        def _(k=k):
            pltpu.sync_copy(i1_hbm, i1_v)
            pltpu.sync_copy(i2_hbm, i2_v)
            hk = i1_v[...][k]
            wk = i2_v[...][k % n_j]
            ht = pl.multiple_of((hk // _BLK) * _BLK, _BLK)
            l = hk - ht
            pltpu.sync_copy(x_hbm.at[wk, :, pl.ds(ht, _BLK)], blk_v)
            b16 = pl.multiple_of((l // _LANES) * _LANES, _LANES)
            lvec = jnp.broadcast_to(l - b16, (_LANES,)).astype(jnp.int32)
            for g in range(d // _LANES):
                acc = jnp.zeros((_LANES,), jnp.float32)
                for c16 in range(_LANES):
                    v16 = blk_v[g * _LANES + c16, pl.ds(b16, _LANES)]
                    acc = jnp.where(lane == c16, _lane_splat(v16, lvec), acc)
                col_v[0, pl.ds(g * _LANES, _LANES)] = acc
            pltpu.sync_copy(col_v, out_hbm.at[k])


def kernel(x, index1, index2):
    h, w, d = x.shape
    n_i, n_j = index1.shape
    n = n_i * n_j
    i1 = jnp.pad(index1.reshape(-1).astype(jnp.int32), (0, _LANES - n))
    i2 = jnp.pad(index2.astype(jnp.int32), (0, _LANES - index2.shape[0]))

    mesh = plsc.VectorSubcoreMesh(core_axis_name="c", subcore_axis_name="s")
    run = pl.kernel(
        functools.partial(_gather_body, n, n_j, d),
        out_type=jax.ShapeDtypeStruct((_LANES, 8, d), jnp.float32),
        mesh=mesh,
        scratch_types=[
            pltpu.VMEM((_LANES,), jnp.int32),
            pltpu.VMEM((_LANES,), jnp.int32),
            pltpu.VMEM((d, _BLK), jnp.float32),
            pltpu.VMEM((8, d), jnp.float32),
        ],
    )
    out16 = run(jnp.transpose(x, (1, 2, 0)), i1, i2)
    return out16[:n, 0, :].reshape(n_i, n_j, d)


# speedup vs baseline: 125.8371x; 1.0279x over previous
import functools

import jax
import jax.numpy as jnp
from jax import lax
from jax.experimental import pallas as pl
from jax.experimental.pallas import tpu as pltpu
from jax.experimental.pallas import tpu_sc as plsc

_LANES = 16
_BLK = 128

_DNUMS = lax.GatherDimensionNumbers(
    offset_dims=(), collapsed_slice_dims=(0,), start_index_map=(0,))


def _lane_splat(vec, idx):
    return lax.gather(vec, idx[:, None], _DNUMS, slice_sizes=(1,),
                      mode=lax.GatherScatterMode.PROMISE_IN_BOUNDS)


def _gather_body(n, n_j, d, x_hbm, idx_hbm, out_hbm, idx_v, blk_v, col_v):
    cid = lax.axis_index("c")
    sid = lax.axis_index("s")
    lane = lax.iota(jnp.int32, _LANES)

    for k in range(n):

        @pl.when(jnp.logical_and(cid == 0, sid == k))
        def _(k=k):
            pltpu.sync_copy(idx_hbm, idx_v)
            hk = idx_v[...][k]
            wk = idx_v[...][n + k % n_j]
            ht = pl.multiple_of((hk // _BLK) * _BLK, _BLK)
            l = hk - ht
            pltpu.sync_copy(x_hbm.at[wk, :, pl.ds(ht, _BLK)], blk_v)
            b16 = pl.multiple_of((l // _LANES) * _LANES, _LANES)
            lvec = jnp.broadcast_to(l - b16, (_LANES,)).astype(jnp.int32)
            for g in range(d // _LANES):
                acc = jnp.zeros((_LANES,), jnp.float32)
                for c16 in range(_LANES):
                    v16 = blk_v[g * _LANES + c16, pl.ds(b16, _LANES)]
                    acc = jnp.where(lane == c16, _lane_splat(v16, lvec), acc)
                col_v[0, pl.ds(g * _LANES, _LANES)] = acc
            pltpu.sync_copy(col_v, out_hbm.at[k])


def kernel(x, index1, index2):
    h, w, d = x.shape
    n_i, n_j = index1.shape
    n = n_i * n_j
    idx = jnp.concatenate([index1.reshape(-1).astype(jnp.int32),
                           index2.astype(jnp.int32)])
    idx = jnp.pad(idx, (0, _LANES - n - n_j))

    mesh = plsc.VectorSubcoreMesh(core_axis_name="c", subcore_axis_name="s")
    run = pl.kernel(
        functools.partial(_gather_body, n, n_j, d),
        out_type=jax.ShapeDtypeStruct((_LANES, 8, d), jnp.float32),
        mesh=mesh,
        scratch_types=[
            pltpu.VMEM((_LANES,), jnp.int32),
            pltpu.VMEM((d, _BLK), jnp.float32),
            pltpu.VMEM((8, d), jnp.float32),
        ],
    )
    out16 = run(jnp.transpose(x, (1, 2, 0)), idx)
    return out16[:n, 0, :].reshape(n_i, n_j, d)
